# Initial kernel scaffold; baseline (speedup 1.0000x reference)
import jax
import jax.numpy as jnp
from jax.experimental import pallas as pl


def kernel(z, pos, edge_index, batch, emb, l0_msg_w1, l0_msg_b1, l0_msg_w2, l0_msg_b2, l0_upd_w1, l0_upd_b1, l0_upd_w2, l0_upd_b2, l1_msg_w1, l1_msg_b1, l1_msg_w2, l1_msg_b2, l1_upd_w1, l1_upd_b1, l1_upd_w2, l1_upd_b2, final_w, final_b):
    raise NotImplementedError("write your pallas kernel here")


# speedup vs baseline: 2.2185x; 2.2185x over previous
import functools

import jax
import jax.numpy as jnp
from jax import lax
from jax.experimental import pallas as pl
from jax.experimental.pallas import tpu as pltpu
from jax.experimental.pallas import tpu_sc as plsc

H = 128
NFD = 512
N = 10000
E = 320000
NG = 64
EMB = 100
F32 = jnp.float32

NC, NS = 2, 16
NW = NC * NS
EPW = E // NW
CE = 80
NCHUNK = EPW // CE

CES = 160
EPT = E // NS
NSCH = EPT // CES
NPAD = 10240
RPT = NPAD // NS
ZR = 128
SPC = (NFD // H) // NC

BN = 1000
BE = 512


def _sigm(x):
    return 1.0 / (1.0 + jnp.exp(-x))


def _edge_gather(P, Q, ppx, ppy, ppz, row, col):
    mesh = plsc.VectorSubcoreMesh(core_axis_name="c", subcore_axis_name="s", num_cores=NC, num_subcores=NS)

    @functools.partial(
        pl.kernel,
        out_type=(jax.ShapeDtypeStruct((E, NFD), F32),
                  jax.ShapeDtypeStruct((E,), F32)),
        mesh=mesh,
        compiler_params=pltpu.CompilerParams(needs_layout_passes=False),
        scratch_types=[
            pltpu.VMEM((N,), F32), pltpu.VMEM((N,), F32), pltpu.VMEM((N,), F32),
            pltpu.VMEM((CE,), jnp.int32), pltpu.VMEM((CE,), jnp.int32),
            pltpu.VMEM((CE, NFD), F32), pltpu.VMEM((CE, NFD), F32),
            pltpu.VMEM((CE,), F32),
            pltpu.SemaphoreType.DMA, pltpu.SemaphoreType.DMA,
        ],
    )
    def k(P_h, Q_h, px_h, py_h, pz_h, row_h, col_h, t_h, d2_h,
          px_v, py_v, pz_v, row_v, col_v, pr_v, qc_v, d2_v, sem1, sem2):
        wid = lax.axis_index("s") * NC + lax.axis_index("c")
        pltpu.sync_copy(px_h, px_v)
        pltpu.sync_copy(py_h, py_v)
        pltpu.sync_copy(pz_h, pz_v)
        base = wid * EPW

        def chunk(c, carry):
            e0 = base + c * CE
            pltpu.sync_copy(row_h.at[pl.ds(e0, CE)], row_v)
            pltpu.sync_copy(col_h.at[pl.ds(e0, CE)], col_v)
            cp1 = pltpu.async_copy(P_h.at[row_v], pr_v, sem1)
            cp2 = pltpu.async_copy(Q_h.at[col_v], qc_v, sem2)
            cp1.wait()
            cp2.wait()

            def rbody(i, cr):
                for j in range(NFD // 16):
                    sl = pl.ds(j * 16, 16)
                    pr_v[i, sl] = pr_v[i, sl] + qc_v[i, sl]
                return cr

            lax.fori_loop(0, CE, rbody, 0)
            for g in range(CE // 16):
                sl = pl.ds(g * 16, 16)
                r16 = row_v[sl]
                c16 = col_v[sl]
                dx = plsc.load_gather(px_v, [r16]) - plsc.load_gather(px_v, [c16])
                dy = plsc.load_gather(py_v, [r16]) - plsc.load_gather(py_v, [c16])
                dz = plsc.load_gather(pz_v, [r16]) - plsc.load_gather(pz_v, [c16])
                d2_v[sl] = dx * dx + dy * dy + dz * dz
            pltpu.sync_copy(pr_v, t_h.at[pl.ds(e0, CE)])
            pltpu.sync_copy(d2_v, d2_h.at[pl.ds(e0, CE)])
            return carry

        lax.fori_loop(0, NCHUNK, chunk, 0)

    return k(P, Q, ppx, ppy, ppz, row, col)


def _scatter_agg(m, row):
    mesh = plsc.VectorSubcoreMesh(core_axis_name="c", subcore_axis_name="s", num_cores=NC, num_subcores=NS)

    @functools.partial(
        pl.kernel,
        out_type=jax.ShapeDtypeStruct((NPAD, NFD), F32),
        mesh=mesh,
        scratch_types=[
            pltpu.VMEM_SHARED((NPAD, H), F32),
            pltpu.VMEM((CES, H), F32),
            pltpu.VMEM((CES,), jnp.int32),
            pltpu.VMEM((ZR, H), F32),
        ],
    )
    def k(m_h, row_h, agg_h, acc_s, buf_v, idx_v, stage_v):
        cid = lax.axis_index("c")
        sid = lax.axis_index("s")
        r0 = sid * RPT

        def zrow(i, cr):
            for j in range(H // 16):
                stage_v[i, pl.ds(j * 16, 16)] = jnp.zeros((16,), F32)
            return cr

        lax.fori_loop(0, ZR, zrow, 0)
        for s in range(SPC):
            f0 = (cid * SPC + s) * H
            for bz in range(RPT // ZR):
                pltpu.sync_copy(stage_v, acc_s.at[pl.ds(r0 + bz * ZR, ZR)])
            plsc.subcore_barrier()

            def chunk(c, cr):
                e0 = sid * EPT + c * CES
                pltpu.sync_copy(row_h.at[pl.ds(e0, CES)], idx_v)
                pltpu.sync_copy(m_h.at[pl.ds(e0, CES), pl.ds(f0, H)], buf_v)
                pltpu.sync_copy(buf_v, acc_s.at[idx_v], add=True)
                return cr

            lax.fori_loop(0, NSCH, chunk, 0)
            plsc.subcore_barrier()
            for bz in range(RPT // ZR):
                pltpu.sync_copy(acc_s.at[pl.ds(r0 + bz * ZR, ZR)], stage_v)
                pltpu.sync_copy(
                    stage_v, agg_h.at[pl.ds(r0 + bz * ZR, ZR), pl.ds(f0, H)])
                lax.fori_loop(0, ZR, zrow, 0)

    return k(m, row)


def _pre0(z2, pos_pad, W0a, W0b, A, B, S):
    def body(z_ref, pos_ref, w0a, w0b, a_ref, b_ref, s_ref,
             nf_ref, p_ref, q_ref, pp_ref):
        zz = z_ref[...]
        iot = lax.broadcasted_iota(jnp.int32, (BN, H), 1)
        oh = (iot == zz).astype(F32)
        nf = (jnp.dot(oh, w0a[...], preferred_element_type=F32)
              + jnp.dot(pos_ref[...], w0b[...], preferred_element_type=F32))
        nf_ref[...] = nf
        p_ref[...] = jnp.dot(nf, a_ref[...], preferred_element_type=F32)
        q_ref[...] = jnp.dot(nf, b_ref[...], preferred_element_type=F32)
        pp_ref[...] = jnp.dot(nf, s_ref[...], preferred_element_type=F32)

    return pl.pallas_call(
        body,
        grid=(N // BN,),
        in_specs=[
            pl.BlockSpec((BN, 1), lambda i: (i, 0)),
            pl.BlockSpec((BN, H), lambda i: (i, 0)),
            pl.BlockSpec((H, NFD), lambda i: (0, 0)),
            pl.BlockSpec((H, NFD), lambda i: (0, 0)),
            pl.BlockSpec((NFD, NFD), lambda i: (0, 0)),
            pl.BlockSpec((NFD, NFD), lambda i: (0, 0)),
            pl.BlockSpec((NFD, H), lambda i: (0, 0)),
        ],
        out_specs=[
            pl.BlockSpec((BN, NFD), lambda i: (i, 0)),
            pl.BlockSpec((BN, NFD), lambda i: (i, 0)),
            pl.BlockSpec((BN, NFD), lambda i: (i, 0)),
            pl.BlockSpec((BN, H), lambda i: (i, 0)),
        ],
        out_shape=[
            jax.ShapeDtypeStruct((N, NFD), F32),
            jax.ShapeDtypeStruct((N, NFD), F32),
            jax.ShapeDtypeStruct((N, NFD), F32),
            jax.ShapeDtypeStruct((N, H), F32),
        ],
    )(z2, pos_pad, W0a, W0b, A, B, S)


def _edge_mm(t, d2c, wd, b1, W2, b2):
    def body(t_ref, d2_ref, wd_ref, b1_ref, w2_ref, b2_ref, m_ref):
        dist = jnp.sqrt(d2_ref[...] + 1e-12)
        h = t_ref[...] + dist * wd_ref[...] + b1_ref[...]
        h = h * _sigm(h)
        mm = jnp.dot(h, w2_ref[...], preferred_element_type=F32) + b2_ref[...]
        m_ref[...] = mm * _sigm(mm)

    return pl.pallas_call(
        body,
        grid=(E // BE,),
        in_specs=[
            pl.BlockSpec((BE, NFD), lambda i: (i, 0)),
            pl.BlockSpec((BE, 1), lambda i: (i, 0)),
            pl.BlockSpec((1, NFD), lambda i: (0, 0)),
            pl.BlockSpec((1, NFD), lambda i: (0, 0)),
            pl.BlockSpec((NFD, NFD), lambda i: (0, 0)),
            pl.BlockSpec((1, NFD), lambda i: (0, 0)),
        ],
        out_specs=pl.BlockSpec((BE, NFD), lambda i: (i, 0)),
        out_shape=jax.ShapeDtypeStruct((E, NFD), F32),
    )(t, d2c, wd, b1, W2, b2)


def _update(nf, agg, U1a, U1b, ub1, uw2, ub2, nxt=None):
    with_next = nxt is not None

    def body(nf_ref, agg_ref, u1a, u1b, ub1_ref, uw2_ref, ub2_ref, *rest):
        if with_next:
            a_ref, b_ref, s_ref, nf2_ref, p_ref, q_ref, pp_ref = rest
        else:
            (nf2_ref,) = rest
        hh = (jnp.dot(nf_ref[...], u1a[...], preferred_element_type=F32)
              + jnp.dot(agg_ref[...], u1b[...], preferred_element_type=F32)
              + ub1_ref[...])
        hh = hh * _sigm(hh)
        new = jnp.dot(hh, uw2_ref[...], preferred_element_type=F32) + ub2_ref[...]
        nf2_ref[...] = new
        if with_next:
            p_ref[...] = jnp.dot(new, a_ref[...], preferred_element_type=F32)
            q_ref[...] = jnp.dot(new, b_ref[...], preferred_element_type=F32)
            pp_ref[...] = jnp.dot(new, s_ref[...], preferred_element_type=F32)

    cst = lambda bs: pl.BlockSpec(bs, lambda i: (0, 0))
    row_spec = lambda w: pl.BlockSpec((BN, w), lambda i: (i, 0))
    in_specs = [row_spec(NFD), row_spec(NFD), cst((NFD, NFD)), cst((NFD, NFD)),
                cst((1, NFD)), cst((NFD, NFD)), cst((1, NFD))]
    out_specs = [row_spec(NFD)]
    out_shape = [jax.ShapeDtypeStruct((N, NFD), F32)]
    args = [nf, agg, U1a, U1b, ub1, uw2, ub2]
    if with_next:
        A2, B2, S = nxt
        in_specs += [cst((NFD, NFD)), cst((NFD, NFD)), cst((NFD, H))]
        out_specs += [row_spec(NFD), row_spec(NFD), row_spec(H)]
        out_shape += [jax.ShapeDtypeStruct((N, NFD), F32),
                      jax.ShapeDtypeStruct((N, NFD), F32),
                      jax.ShapeDtypeStruct((N, H), F32)]
        args += [A2, B2, S]
    return pl.pallas_call(
        body, grid=(N // BN,), in_specs=in_specs,
        out_specs=out_specs, out_shape=out_shape,
    )(*args)


def _pool(nf2, batch2, W3, fb):
    def body(nf_ref, b_ref, w3_ref, fb_ref, out_ref, acc_ref):
        i = pl.program_id(0)
        g = b_ref[...]
        iot = lax.broadcasted_iota(jnp.int32, (BN, H), 1)
        oh = (iot == g).astype(F32)
        part = lax.dot_general(oh, nf_ref[...], (((0,), (0,)), ((), ())),
                               preferred_element_type=F32)

        @pl.when(i == 0)
        def _():
            acc_ref[...] = part

        @pl.when(i > 0)
        def _():
            acc_ref[...] = acc_ref[...] + part

        @pl.when(i == N // BN - 1)
        def _():
            out_ref[...] = (jnp.dot(acc_ref[...], w3_ref[...],
                                    preferred_element_type=F32) + fb_ref[...])

    return pl.pallas_call(
        body,
        grid=(N // BN,),
        in_specs=[
            pl.BlockSpec((BN, NFD), lambda i: (i, 0)),
            pl.BlockSpec((BN, 1), lambda i: (i, 0)),
            pl.BlockSpec((NFD, H), lambda i: (0, 0)),
            pl.BlockSpec((1, H), lambda i: (0, 0)),
        ],
        out_specs=pl.BlockSpec((H, H), lambda i: (0, 0)),
        out_shape=jax.ShapeDtypeStruct((H, H), F32),
        scratch_shapes=[pltpu.VMEM((H, NFD), F32)],
    )(nf2, batch2, W3, fb)


def kernel(z, pos, edge_index, batch, emb,
           l0_msg_w1, l0_msg_b1, l0_msg_w2, l0_msg_b2,
           l0_upd_w1, l0_upd_b1, l0_upd_w2, l0_upd_b2,
           l1_msg_w1, l1_msg_b1, l1_msg_w2, l1_msg_b2,
           l1_upd_w1, l1_upd_b1, l1_upd_w2, l1_upd_b2,
           final_w, final_b):
    row = edge_index[0].astype(jnp.int32)
    col = edge_index[1].astype(jnp.int32)

    W0a = jnp.zeros((H, NFD), F32).at[:EMB, :H].set(emb)
    W0b = jnp.zeros((H, NFD), F32)
    S = jnp.zeros((NFD, H), F32)
    W3 = jnp.zeros((NFD, H), F32)
    for k3 in range(3):
        W0b = W0b.at[k3, H * (k3 + 1):H * (k3 + 2)].set(1.0)
        S = S.at[H * (k3 + 1):H * (k3 + 2), k3].set(1.0 / H)
        W3 = W3.at[H * (k3 + 1):H * (k3 + 2), k3].set(final_w[:, 0])
    fb = jnp.full((1, H), final_b[0], F32)
    pos_pad = jnp.zeros((N, H), F32).at[:, :3].set(pos)
    z2 = z.astype(jnp.int32).reshape(N, 1)
    batch2 = batch.astype(jnp.int32).reshape(N, 1)

    def msg_parts(w1, b1, w2, b2):
        return (w1[:NFD], w1[NFD:2 * NFD], w1[2 * NFD].reshape(1, NFD),
                b1.reshape(1, NFD), w2, b2.reshape(1, NFD))

    A0, B0, wd0, b10, W20, b20 = msg_parts(l0_msg_w1, l0_msg_b1, l0_msg_w2, l0_msg_b2)
    A1, B1, wd1, b11, W21, b21 = msg_parts(l1_msg_w1, l1_msg_b1, l1_msg_w2, l1_msg_b2)

    nf0, P0, Q0, pp0 = _pre0(z2, pos_pad, W0a, W0b, A0, B0, S)

    t0, d20 = _edge_gather(P0, Q0, pp0[:, 0], pp0[:, 1], pp0[:, 2], row, col)
    m0 = _edge_mm(t0, d20.reshape(E, 1), wd0, b10, W20, b20)
    agg0 = _scatter_agg(m0, row)
    nf1, P1, Q1, pp1 = _update(nf0, agg0, l0_upd_w1[:NFD], l0_upd_w1[NFD:],
                               l0_upd_b1.reshape(1, NFD), l0_upd_w2,
                               l0_upd_b2.reshape(1, NFD), nxt=(A1, B1, S))

    t1, d21 = _edge_gather(P1, Q1, pp1[:, 0], pp1[:, 1], pp1[:, 2], row, col)
    m1 = _edge_mm(t1, d21.reshape(E, 1), wd1, b11, W21, b21)
    agg1 = _scatter_agg(m1, row)
    (nf2,) = _update(nf1, agg1, l1_upd_w1[:NFD], l1_upd_w1[NFD:],
                     l1_upd_b1.reshape(1, NFD), l1_upd_w2,
                     l1_upd_b2.reshape(1, NFD))

    out = _pool(nf2, batch2, W3, fb)
    return out[:NG, :3]
